# baseline (speedup 1.0000x reference)
import functools

import jax
import jax.numpy as jnp
from jax import lax
from jax.experimental import pallas as pl
from jax.experimental.pallas import tpu as pltpu
from jax.experimental.pallas import tpu_sc as plsc

N, K, D = 4096, 8192, 256
BN = 256
NB = N // BN
IOTA_BIAS = 0x4B000000


def _argmin_body(x_ref, cb_ref, xsq_ref, csq_ref, idx_ref, cross_ref):
    i = pl.program_id(0)
    par = lax.rem(i, 2)
    prev = lax.rem(i + 1, 2)

    x2 = x_ref[...] + x_ref[...]
    cross_ref[pl.ds(par * BN, BN), :] = lax.dot_general(
        x2, cb_ref[...], (((1,), (1,)), ((), ())),
        preferred_element_type=jnp.float32)

    cross2 = cross_ref[pl.ds(prev * BN, BN), :]
    raw = (xsq_ref[...] + csq_ref[...]) - cross2
    m = jnp.maximum(jnp.min(raw, axis=1, keepdims=True), 1e-12)

    s = jnp.sqrt(m)
    mbits = lax.bitcast_convert_type(m, jnp.int32)
    thr = m
    for j in (1, 2, 3, 4):
        cj = lax.bitcast_convert_type(mbits + j, jnp.float32)
        thr = jnp.where(jnp.sqrt(cj) == s, cj, thr)

    io = lax.broadcasted_iota(jnp.int32, (BN, K), 1) + IOTA_BIAS
    io_f = lax.bitcast_convert_type(io, jnp.float32)
    cand = jnp.where(raw <= thr, io_f, jnp.float32(2.0 ** 23 + K))
    bestf = jnp.min(cand, axis=1, keepdims=True)
    idx_ref[...] = lax.bitcast_convert_type(bestf, jnp.int32) - IOTA_BIAS


def _tc_argmin(x, codebook, interpret=False):
    x_sq = jnp.sum(x * x, axis=-1, keepdims=True)
    c_sq = jnp.sum(codebook * codebook, axis=-1)[None, :]
    return pl.pallas_call(
        _argmin_body,
        grid=(NB + 1,),
        in_specs=[
            pl.BlockSpec((BN, D), lambda i: (jnp.minimum(i, NB - 1), 0)),
            pl.BlockSpec((K, D), lambda i: (0, 0)),
            pl.BlockSpec((BN, 1), lambda i: (jnp.maximum(i - 1, 0), 0)),
            pl.BlockSpec((1, K), lambda i: (0, 0)),
        ],
        out_specs=pl.BlockSpec((BN, 1), lambda i: (jnp.maximum(i - 1, 0), 0)),
        out_shape=jax.ShapeDtypeStruct((N, 1), jnp.int32),
        scratch_shapes=[pltpu.VMEM((2 * BN, K), jnp.float32)],
        compiler_params=pltpu.CompilerParams(
            dimension_semantics=("arbitrary",)),
        interpret=interpret,
    )(x, codebook, x_sq, c_sq)


@functools.lru_cache(maxsize=None)
def _make_sc_gather():
    info = plsc.get_sparse_core_info()
    nw = info.num_cores * info.num_subcores
    bpw = N // nw
    mesh = plsc.VectorSubcoreMesh(core_axis_name="c", subcore_axis_name="s")

    @functools.partial(
        pl.kernel,
        mesh=mesh,
        out_type=jax.ShapeDtypeStruct((N, D), jnp.float32),
        scratch_types=[
            pltpu.VMEM((bpw,), jnp.int32),
            pltpu.VMEM((bpw, D), jnp.float32),
            pltpu.SemaphoreType.DMA,
        ],
    )
    def _sc_gather(table_hbm, idx_hbm, out_hbm, idx_v, rows_v, sem):
        wid = lax.axis_index("s") * info.num_cores + lax.axis_index("c")
        base = wid * bpw
        pltpu.sync_copy(idx_hbm.at[pl.ds(base, bpw)], idx_v)
        pltpu.async_copy(table_hbm.at[idx_v], rows_v, sem).wait()
        pltpu.sync_copy(rows_v, out_hbm.at[pl.ds(base, bpw)])

    return _sc_gather


def kernel(x, codebook, embedding_table):
    idx = _tc_argmin(x, codebook).reshape(N)
    return _make_sc_gather()(embedding_table, idx)


# speedup vs baseline: 1.4453x; 1.4453x over previous
import functools

import jax
import jax.numpy as jnp
from jax import lax
from jax.experimental import pallas as pl
from jax.experimental.pallas import tpu as pltpu
from jax.experimental.pallas import tpu_sc as plsc

N, K, D = 4096, 8192, 256
BN = 256
NB = N // BN
IOTA_BIAS = 0x4B000000


def _argmin_body(x_ref, cb_ref, xsq_ref, csq_ref, idx_ref):
    x2 = x_ref[...] + x_ref[...]
    cross2 = lax.dot_general(
        x2, cb_ref[...], (((1,), (1,)), ((), ())),
        preferred_element_type=jnp.float32)
    raw = (xsq_ref[...] + csq_ref[...]) - cross2
    m = jnp.maximum(jnp.min(raw, axis=1, keepdims=True), 1e-12)

    s = jnp.sqrt(m)
    mbits = lax.bitcast_convert_type(m, jnp.int32)
    thr = m
    for j in (1, 2, 3, 4):
        cj = lax.bitcast_convert_type(mbits + j, jnp.float32)
        thr = jnp.where(jnp.sqrt(cj) == s, cj, thr)

    io = lax.broadcasted_iota(jnp.int32, (BN, K), 1) + IOTA_BIAS
    io_f = lax.bitcast_convert_type(io, jnp.float32)
    cand = jnp.where(raw <= thr, io_f, jnp.float32(2.0 ** 23 + K))
    bestf = jnp.min(cand, axis=1, keepdims=True)
    idx_ref[...] = lax.bitcast_convert_type(bestf, jnp.int32) - IOTA_BIAS


def _tc_argmin(x, codebook, interpret=False):
    x_sq = jnp.sum(x * x, axis=-1, keepdims=True)
    c_sq = jnp.sum(codebook * codebook, axis=-1)[None, :]
    return pl.pallas_call(
        _argmin_body,
        grid=(NB,),
        in_specs=[
            pl.BlockSpec((BN, D), lambda i: (i, 0)),
            pl.BlockSpec((K, D), lambda i: (0, 0)),
            pl.BlockSpec((BN, 1), lambda i: (i, 0)),
            pl.BlockSpec((1, K), lambda i: (0, 0)),
        ],
        out_specs=pl.BlockSpec((BN, 1), lambda i: (i, 0)),
        out_shape=jax.ShapeDtypeStruct((N, 1), jnp.int32),
        compiler_params=pltpu.CompilerParams(
            dimension_semantics=("arbitrary",)),
        interpret=interpret,
    )(x, codebook, x_sq, c_sq)


@functools.lru_cache(maxsize=None)
def _make_sc_gather():
    info = plsc.get_sparse_core_info()
    nw = info.num_cores * info.num_subcores
    bpw = N // nw
    mesh = plsc.VectorSubcoreMesh(core_axis_name="c", subcore_axis_name="s")

    @functools.partial(
        pl.kernel,
        mesh=mesh,
        out_type=jax.ShapeDtypeStruct((N, D), jnp.float32),
        scratch_types=[
            pltpu.VMEM((bpw,), jnp.int32),
            pltpu.VMEM((bpw, D), jnp.float32),
            pltpu.SemaphoreType.DMA,
        ],
    )
    def _sc_gather(table_hbm, idx_hbm, out_hbm, idx_v, rows_v, sem):
        wid = lax.axis_index("s") * info.num_cores + lax.axis_index("c")
        base = wid * bpw
        pltpu.sync_copy(idx_hbm.at[pl.ds(base, bpw)], idx_v)
        pltpu.async_copy(table_hbm.at[idx_v], rows_v, sem).wait()
        pltpu.sync_copy(rows_v, out_hbm.at[pl.ds(base, bpw)])

    return _sc_gather


def kernel(x, codebook, embedding_table):
    idx = _tc_argmin(x, codebook).reshape(N)
    return _make_sc_gather()(embedding_table, idx)


# speedup vs baseline: 1.4730x; 1.0192x over previous
import functools

import jax
import jax.numpy as jnp
from jax import lax
from jax.experimental import pallas as pl
from jax.experimental.pallas import tpu as pltpu
from jax.experimental.pallas import tpu_sc as plsc

N, K, D = 4096, 8192, 256
BN = 512
NB = N // BN
IOTA_BIAS = 0x4B000000


def _argmin_body(x_ref, cb_ref, xsq_ref, csq_ref, idx_ref):
    x2 = x_ref[...] + x_ref[...]
    cross2 = lax.dot_general(
        x2, cb_ref[...], (((1,), (1,)), ((), ())),
        preferred_element_type=jnp.float32)
    raw = (xsq_ref[...] + csq_ref[...]) - cross2
    m = jnp.maximum(jnp.min(raw, axis=1, keepdims=True), 1e-12)

    s = jnp.sqrt(m)
    mbits = lax.bitcast_convert_type(m, jnp.int32)
    thr = m
    for j in (1, 2, 3, 4):
        cj = lax.bitcast_convert_type(mbits + j, jnp.float32)
        thr = jnp.where(jnp.sqrt(cj) == s, cj, thr)

    io = lax.broadcasted_iota(jnp.int32, (BN, K), 1) + IOTA_BIAS
    io_f = lax.bitcast_convert_type(io, jnp.float32)
    cand = jnp.where(raw <= thr, io_f, jnp.float32(2.0 ** 23 + K))
    bestf = jnp.min(cand, axis=1, keepdims=True)
    idx_ref[...] = lax.bitcast_convert_type(bestf, jnp.int32) - IOTA_BIAS


def _tc_argmin(x, codebook, interpret=False):
    x_sq = jnp.sum(x * x, axis=-1, keepdims=True)
    c_sq = jnp.sum(codebook * codebook, axis=-1)[None, :]
    return pl.pallas_call(
        _argmin_body,
        grid=(NB,),
        in_specs=[
            pl.BlockSpec((BN, D), lambda i: (i, 0)),
            pl.BlockSpec((K, D), lambda i: (0, 0)),
            pl.BlockSpec((BN, 1), lambda i: (i, 0)),
            pl.BlockSpec((1, K), lambda i: (0, 0)),
        ],
        out_specs=pl.BlockSpec((BN, 1), lambda i: (i, 0)),
        out_shape=jax.ShapeDtypeStruct((N, 1), jnp.int32),
        compiler_params=pltpu.CompilerParams(
            dimension_semantics=("arbitrary",)),
        interpret=interpret,
    )(x, codebook, x_sq, c_sq)


@functools.lru_cache(maxsize=None)
def _make_sc_gather():
    info = plsc.get_sparse_core_info()
    nw = info.num_cores * info.num_subcores
    bpw = N // nw
    mesh = plsc.VectorSubcoreMesh(core_axis_name="c", subcore_axis_name="s")

    @functools.partial(
        pl.kernel,
        mesh=mesh,
        out_type=jax.ShapeDtypeStruct((N, D), jnp.float32),
        scratch_types=[
            pltpu.VMEM((bpw,), jnp.int32),
            pltpu.VMEM((bpw, D), jnp.float32),
            pltpu.SemaphoreType.DMA,
        ],
    )
    def _sc_gather(table_hbm, idx_hbm, out_hbm, idx_v, rows_v, sem):
        wid = lax.axis_index("s") * info.num_cores + lax.axis_index("c")
        base = wid * bpw
        pltpu.sync_copy(idx_hbm.at[pl.ds(base, bpw)], idx_v)
        pltpu.async_copy(table_hbm.at[idx_v], rows_v, sem).wait()
        pltpu.sync_copy(rows_v, out_hbm.at[pl.ds(base, bpw)])

    return _sc_gather


def kernel(x, codebook, embedding_table):
    idx = _tc_argmin(x, codebook).reshape(N)
    return _make_sc_gather()(embedding_table, idx)


# speedup vs baseline: 1.5032x; 1.0205x over previous
import functools

import jax
import jax.numpy as jnp
from jax import lax
from jax.experimental import pallas as pl
from jax.experimental.pallas import tpu as pltpu
from jax.experimental.pallas import tpu_sc as plsc

N, K, D = 4096, 8192, 256
BN = 1024
NB = N // BN
IOTA_BIAS = 0x4B000000


def _argmin_body(x_ref, cb_ref, xsq_ref, csq_ref, idx_ref):
    x2 = x_ref[...] + x_ref[...]
    cross2 = lax.dot_general(
        x2, cb_ref[...], (((1,), (1,)), ((), ())),
        preferred_element_type=jnp.float32)
    raw = (xsq_ref[...] + csq_ref[...]) - cross2
    m = jnp.maximum(jnp.min(raw, axis=1, keepdims=True), 1e-12)

    s = jnp.sqrt(m)
    mbits = lax.bitcast_convert_type(m, jnp.int32)
    thr = m
    for j in (1, 2, 3, 4):
        cj = lax.bitcast_convert_type(mbits + j, jnp.float32)
        thr = jnp.where(jnp.sqrt(cj) == s, cj, thr)

    io = lax.broadcasted_iota(jnp.int32, (BN, K), 1) + IOTA_BIAS
    io_f = lax.bitcast_convert_type(io, jnp.float32)
    cand = jnp.where(raw <= thr, io_f, jnp.float32(2.0 ** 23 + K))
    bestf = jnp.min(cand, axis=1, keepdims=True)
    idx_ref[...] = lax.bitcast_convert_type(bestf, jnp.int32) - IOTA_BIAS


def _tc_argmin(x, codebook, interpret=False):
    x_sq = jnp.sum(x * x, axis=-1, keepdims=True)
    c_sq = jnp.sum(codebook * codebook, axis=-1)[None, :]
    return pl.pallas_call(
        _argmin_body,
        grid=(NB,),
        in_specs=[
            pl.BlockSpec((BN, D), lambda i: (i, 0)),
            pl.BlockSpec((K, D), lambda i: (0, 0)),
            pl.BlockSpec((BN, 1), lambda i: (i, 0)),
            pl.BlockSpec((1, K), lambda i: (0, 0)),
        ],
        out_specs=pl.BlockSpec((BN, 1), lambda i: (i, 0)),
        out_shape=jax.ShapeDtypeStruct((N, 1), jnp.int32),
        compiler_params=pltpu.CompilerParams(
            dimension_semantics=("arbitrary",)),
        interpret=interpret,
    )(x, codebook, x_sq, c_sq)


@functools.lru_cache(maxsize=None)
def _make_sc_gather():
    info = plsc.get_sparse_core_info()
    nw = info.num_cores * info.num_subcores
    bpw = N // nw
    mesh = plsc.VectorSubcoreMesh(core_axis_name="c", subcore_axis_name="s")

    @functools.partial(
        pl.kernel,
        mesh=mesh,
        out_type=jax.ShapeDtypeStruct((N, D), jnp.float32),
        scratch_types=[
            pltpu.VMEM((bpw,), jnp.int32),
            pltpu.VMEM((bpw, D), jnp.float32),
            pltpu.SemaphoreType.DMA,
        ],
    )
    def _sc_gather(table_hbm, idx_hbm, out_hbm, idx_v, rows_v, sem):
        wid = lax.axis_index("s") * info.num_cores + lax.axis_index("c")
        base = wid * bpw
        pltpu.sync_copy(idx_hbm.at[pl.ds(base, bpw)], idx_v)
        pltpu.async_copy(table_hbm.at[idx_v], rows_v, sem).wait()
        pltpu.sync_copy(rows_v, out_hbm.at[pl.ds(base, bpw)])

    return _sc_gather


def kernel(x, codebook, embedding_table):
    idx = _tc_argmin(x, codebook).reshape(N)
    return _make_sc_gather()(embedding_table, idx)
